# baseline (speedup 1.0000x reference)
import functools

import jax
import jax.numpy as jnp
from jax import lax
from jax.experimental import pallas as pl
from jax.experimental.pallas import tpu as pltpu
import jax.experimental.pallas.tpu_sc as plsc

N = 10000
D = 128
E = 320000
ETOT = E + N

NW = 32
CHUNK = 64
NCH = 162
IDXR = 81
PER_TILE = NCH * CHUNK
EPAD = NW * PER_TILE
PAD = EPAD - ETOT

NPAD = 10112
GARBAGE = 10016
RPT = NPAD // 16
ZN = 640

BN = 1000
NBLK = N // BN


def _pre_body(act_ref, w_ref, atts_ref, attd_ref,
              h_ref, asrc_ref, adst_ref, c_ref, ms_ref, md_ref):
    i = pl.program_id(0)
    h = jnp.dot(act_ref[...], w_ref[...], preferred_element_type=jnp.float32)
    h_ref[...] = h.astype(jnp.bfloat16)
    asrc = jnp.dot(h, atts_ref[...], preferred_element_type=jnp.float32)
    adst = jnp.dot(h, attd_ref[...], preferred_element_type=jnp.float32)
    asrc_ref[...] = asrc
    adst_ref[...] = adst

    @pl.when(i == 0)
    def _():
        ms_ref[...] = jnp.full((8, 128), -1e30, jnp.float32)
        md_ref[...] = jnp.full((8, 128), -1e30, jnp.float32)

    ms_ref[...] = jnp.maximum(ms_ref[...], jnp.max(asrc))
    md_ref[...] = jnp.maximum(md_ref[...], jnp.max(adst))

    @pl.when(i == NBLK - 1)
    def _():
        c_ref[...] = jnp.maximum(ms_ref[...] + md_ref[...], 0.0)


def _pre(act, w, atts, attd):
    return pl.pallas_call(
        _pre_body,
        grid=(NBLK,),
        in_specs=[
            pl.BlockSpec((BN, D), lambda i: (i, 0)),
            pl.BlockSpec((D, D), lambda i: (0, 0)),
            pl.BlockSpec((D, 1), lambda i: (0, 0)),
            pl.BlockSpec((D, 1), lambda i: (0, 0)),
        ],
        out_specs=[
            pl.BlockSpec((BN, D), lambda i: (i, 0)),
            pl.BlockSpec((BN, 1), lambda i: (i, 0)),
            pl.BlockSpec((BN, 1), lambda i: (i, 0)),
            pl.BlockSpec((8, 128), lambda i: (0, 0)),
        ],
        out_shape=[
            jax.ShapeDtypeStruct((N, D), jnp.bfloat16),
            jax.ShapeDtypeStruct((N, 1), jnp.float32),
            jax.ShapeDtypeStruct((N, 1), jnp.float32),
            jax.ShapeDtypeStruct((8, 128), jnp.float32),
        ],
        scratch_shapes=[
            pltpu.VMEM((8, 128), jnp.float32),
            pltpu.VMEM((8, 128), jnp.float32),
        ],
    )(act, w, atts, attd)


def _mid_body(n0_ref, n1_ref, d0_ref, d1_ref, b_ref, w_ref, atts_ref, attd_ref,
              h_ref, asrc_ref, adst_ref, c_ref, ms_ref, md_ref):
    i = pl.program_id(0)
    d = d0_ref[...] + d1_ref[...]
    d = jnp.where(d == 0.0, 1.0, d)
    act = (n0_ref[...] + n1_ref[...]) / d + b_ref[...][0:1, :]
    act = jnp.maximum(act, 0.0)
    h = jnp.dot(act, w_ref[...], preferred_element_type=jnp.float32)
    h_ref[...] = h.astype(jnp.bfloat16)
    asrc = jnp.dot(h, atts_ref[...], preferred_element_type=jnp.float32)
    adst = jnp.dot(h, attd_ref[...], preferred_element_type=jnp.float32)
    asrc_ref[...] = asrc
    adst_ref[...] = adst

    @pl.when(i == 0)
    def _():
        ms_ref[...] = jnp.full((8, 128), -1e30, jnp.float32)
        md_ref[...] = jnp.full((8, 128), -1e30, jnp.float32)

    ms_ref[...] = jnp.maximum(ms_ref[...], jnp.max(asrc))
    md_ref[...] = jnp.maximum(md_ref[...], jnp.max(adst))

    @pl.when(i == NBLK - 1)
    def _():
        c_ref[...] = jnp.maximum(ms_ref[...] + md_ref[...], 0.0)


def _mid(n0, n1, d0, d1, b8, w, atts, attd):
    return pl.pallas_call(
        _mid_body,
        grid=(NBLK,),
        in_specs=[
            pl.BlockSpec((BN, D), lambda i: (i, 0)),
            pl.BlockSpec((BN, D), lambda i: (i, 0)),
            pl.BlockSpec((BN, 1), lambda i: (i, 0)),
            pl.BlockSpec((BN, 1), lambda i: (i, 0)),
            pl.BlockSpec((8, 128), lambda i: (0, 0)),
            pl.BlockSpec((D, D), lambda i: (0, 0)),
            pl.BlockSpec((D, 1), lambda i: (0, 0)),
            pl.BlockSpec((D, 1), lambda i: (0, 0)),
        ],
        out_specs=[
            pl.BlockSpec((BN, D), lambda i: (i, 0)),
            pl.BlockSpec((BN, 1), lambda i: (i, 0)),
            pl.BlockSpec((BN, 1), lambda i: (i, 0)),
            pl.BlockSpec((8, 128), lambda i: (0, 0)),
        ],
        out_shape=[
            jax.ShapeDtypeStruct((N, D), jnp.bfloat16),
            jax.ShapeDtypeStruct((N, 1), jnp.float32),
            jax.ShapeDtypeStruct((N, 1), jnp.float32),
            jax.ShapeDtypeStruct((8, 128), jnp.float32),
        ],
        scratch_shapes=[
            pltpu.VMEM((8, 128), jnp.float32),
            pltpu.VMEM((8, 128), jnp.float32),
        ],
    )(n0, n1, d0, d1, b8, w, atts, attd)


def _norm_body(n0_ref, n1_ref, d0_ref, d1_ref, b_ref, out_ref):
    d = d0_ref[...] + d1_ref[...]
    d = jnp.where(d == 0.0, 1.0, d)
    a = (n0_ref[...] + n1_ref[...]) / d + b_ref[...][0:1, :]
    out_ref[...] = jnp.maximum(a, 0.0)


def _norm(n0, n1, d0, d1, b8):
    return pl.pallas_call(
        _norm_body,
        grid=(NBLK,),
        in_specs=[
            pl.BlockSpec((BN, D), lambda i: (i, 0)),
            pl.BlockSpec((BN, D), lambda i: (i, 0)),
            pl.BlockSpec((BN, 1), lambda i: (i, 0)),
            pl.BlockSpec((BN, 1), lambda i: (i, 0)),
            pl.BlockSpec((8, 128), lambda i: (0, 0)),
        ],
        out_specs=pl.BlockSpec((BN, D), lambda i: (i, 0)),
        out_shape=jax.ShapeDtypeStruct((N, D), jnp.float32),
    )(n0, n1, d0, d1, b8)


_MESH = plsc.VectorSubcoreMesh(core_axis_name="c", subcore_axis_name="s")


@functools.partial(
    pl.kernel,
    out_type=[
        jax.ShapeDtypeStruct((2 * NPAD, D), jnp.float32),
        jax.ShapeDtypeStruct((2 * NPAD,), jnp.float32),
    ],
    mesh=_MESH,
    scratch_types=[
        pltpu.VMEM_SHARED((NPAD, D), jnp.float32),
        pltpu.VMEM_SHARED((NPAD,), jnp.float32),
        pltpu.VMEM((IDXR, 128), jnp.int32),
        pltpu.VMEM((IDXR, 128), jnp.int32),
        pltpu.VMEM((2, CHUNK), jnp.int32),
        pltpu.VMEM((2, CHUNK), jnp.float32),
        pltpu.VMEM((2, CHUNK), jnp.float32),
        pltpu.VMEM((2, CHUNK), jnp.float32),
        pltpu.VMEM((128,), jnp.float32),
        pltpu.VMEM((CHUNK, D // 2), jnp.int32),
        pltpu.VMEM((CHUNK, D // 2), jnp.int32),
        pltpu.VMEM((CHUNK, D), jnp.float32),
        pltpu.VMEM((CHUNK, D), jnp.float32),
        pltpu.SemaphoreType.DMA,
        pltpu.SemaphoreType.DMA,
        pltpu.SemaphoreType.DMA,
        pltpu.SemaphoreType.DMA,
        pltpu.SemaphoreType.DMA,
        pltpu.SemaphoreType.DMA,
    ],
    compiler_params=pltpu.CompilerParams(needs_layout_passes=False,
                                         use_tc_tiling_on_sc=False),
)
def _edge(h_hbm, src_hbm, dst_hbm, asrc_hbm, adst_hbm, c_hbm,
          num_hbm, den_hbm,
          num_s, den_s, src2d, dst2d, didx2, ex2, av2, bv2, cbuf,
          rowsb0, rowsb1, rowf0, rowf1,
          sg0, sg1, ss0, ss1, sd0, sd1):
    c = lax.axis_index("c")
    s = lax.axis_index("s")
    w = c * 16 + s
    rowsb = [rowsb0, rowsb1]
    rowf = [rowf0, rowf1]
    sg = [sg0, sg1]
    ss = [ss0, ss1]
    sd = [sd0, sd1]

    pltpu.async_copy(src_hbm.at[w], src2d, sg1)
    pltpu.async_copy(dst_hbm.at[w], dst2d, sg1)
    pltpu.async_copy(c_hbm.at[0], cbuf, sg1)

    def _zrow(r, _):
        for j in range(8):
            rowf0[r, pl.ds(j * 16, 16)] = jnp.zeros((16,), jnp.float32)
        return 0
    lax.fori_loop(0, CHUNK, _zrow, 0)
    for g in range(CHUNK // 16):
        av2[0, pl.ds(g * 16, 16)] = jnp.zeros((16,), jnp.float32)

    base = s * RPT
    nzc = RPT // CHUNK
    rem = RPT - nzc * CHUNK
    for k in range(nzc):
        pltpu.async_copy(rowf0, num_s.at[pl.ds(base + k * CHUNK, CHUNK)], sg0)
        pltpu.async_copy(av2.at[0], den_s.at[pl.ds(base + k * CHUNK, CHUNK)],
                         sg0)
    pltpu.async_copy(rowf0.at[pl.ds(0, rem)],
                     num_s.at[pl.ds(base + nzc * CHUNK, rem)], sg0)
    pltpu.async_copy(av2.at[0, pl.ds(0, rem)],
                     den_s.at[pl.ds(base + nzc * CHUNK, rem)], sg0)
    for k in range(nzc):
        pltpu.make_async_copy(
            rowf0, num_s.at[pl.ds(base + k * CHUNK, CHUNK)], sg0).wait()
        pltpu.make_async_copy(
            av2.at[0], den_s.at[pl.ds(base + k * CHUNK, CHUNK)], sg0).wait()
    pltpu.make_async_copy(
        rowf0.at[pl.ds(0, rem)],
        num_s.at[pl.ds(base + nzc * CHUNK, rem)], sg0).wait()
    pltpu.make_async_copy(
        av2.at[0, pl.ds(0, rem)],
        den_s.at[pl.ds(base + nzc * CHUNK, rem)], sg0).wait()
    pltpu.make_async_copy(src_hbm.at[w], src2d, sg1).wait()
    pltpu.make_async_copy(dst_hbm.at[w], dst2d, sg1).wait()
    pltpu.make_async_copy(c_hbm.at[0], cbuf, sg1).wait()
    plsc.subcore_barrier()

    cvec = cbuf[pl.ds(0, 16)]

    def _sidx(jh, half):
        return src2d.at[jh, pl.ds(half * CHUNK, CHUNK)]

    def _didx(jh, half):
        return dst2d.at[jh, pl.ds(half * CHUNK, CHUNK)]

    idx_e = jnp.arange(0, 32, 2, dtype=jnp.int32)
    idx_o = idx_e + 1

    def _issue(jh, half, b):
        pltpu.async_copy(h_hbm.at[_sidx(jh, half)], rowsb[b], sg[b])
        pltpu.async_copy(asrc_hbm.at[_sidx(jh, half)], av2.at[b], sg[b])
        pltpu.async_copy(adst_hbm.at[_didx(jh, half)], bv2.at[b], sg[b])

    def _wait_gathers(b):
        pltpu.make_async_copy(h_hbm.at[_sidx(0, 0)], rowsb[b], sg[b]).wait()
        pltpu.make_async_copy(asrc_hbm.at[_sidx(0, 0)], av2.at[b], sg[b]).wait()
        pltpu.make_async_copy(adst_hbm.at[_didx(0, 0)], bv2.at[b], sg[b]).wait()

    def _wait_num_scatter(b):
        pltpu.make_async_copy(rowf[b], num_s.at[didx2.at[0]], ss[b]).wait()

    def _wait_den_scatter(b):
        pltpu.make_async_copy(ex2.at[b], den_s.at[didx2.at[0]], sd[b]).wait()

    def _consume(jh, half, b, first):
        _wait_gathers(b)
        if not first:
            _wait_den_scatter(b)
            _wait_num_scatter(b)
        for g in range(CHUNK // 16):
            d16 = dst2d[jh, pl.ds(half * CHUNK + g * 16, 16)]
            didx2[b, pl.ds(g * 16, 16)] = d16
            avv = av2[b, pl.ds(g * 16, 16)]
            bvv = bv2[b, pl.ds(g * 16, 16)]
            al = avv + bvv
            al = jnp.maximum(al, al * jnp.float32(0.2))
            ex2[b, pl.ds(g * 16, 16)] = jnp.exp(al - cvec)
        pltpu.async_copy(ex2.at[b], den_s.at[didx2.at[b]], sd[b], add=True)

        rbb = rowsb[b]
        rbf = rowf[b]
        def _scale(r4, _):
            r0 = r4 * 2
            for u in range(2):
                r = r0 + u
                r16 = jnp.full((16,), r, jnp.int32)
                eb = plsc.load_gather(
                    ex2, [jnp.full((16,), b, jnp.int32), r16])
                for j2 in range(4):
                    packed = plsc.bitcast(rbb[r, pl.ds(j2 * 16, 16)],
                                          jnp.bfloat16)
                    pa, pb = plsc.unpack(
                        packed, format=plsc.PackFormat.INTERLEAVED)
                    plsc.store_scatter(rbf, [r16, idx_e + j2 * 32], pa * eb)
                    plsc.store_scatter(rbf, [r16, idx_o + j2 * 32], pb * eb)
            return 0
        lax.fori_loop(0, CHUNK // 2, _scale, 0)
        pltpu.async_copy(rbf, num_s.at[didx2.at[b]], ss[b], add=True)
        nh = jh + 1
        @pl.when(nh < IDXR)
        def _():
            _issue(nh, half, b)

    _issue(0, 0, 0)
    _issue(0, 1, 1)

    def _superstep(k, _):
        for t in range(2):
            _consume(k, t, t, False)
        return 0
    _consume(0, 0, 0, True)
    _consume(0, 1, 1, True)
    lax.fori_loop(1, IDXR, _superstep, 0)

    for b in range(2):
        _wait_den_scatter(b)
        _wait_num_scatter(b)
    plsc.subcore_barrier()

    hb = c * NPAD + base
    for k in range(nzc + 1):
        rn = CHUNK if k < nzc else rem
        rb = rowf[k % 2]
        eb = ex2 if k % 2 == 0 else bv2
        if k >= 2:
            pltpu.make_async_copy(
                rb, num_hbm.at[pl.ds(hb + (k - 2) * CHUNK, CHUNK)],
                ss[k % 2]).wait()
            pltpu.make_async_copy(
                eb.at[0], den_hbm.at[pl.ds(hb + (k - 2) * CHUNK, CHUNK)],
                sg[k % 2]).wait()
        pltpu.sync_copy(num_s.at[pl.ds(base + k * CHUNK, rn)],
                        rb.at[pl.ds(0, rn)])
        pltpu.async_copy(rb.at[pl.ds(0, rn)],
                         num_hbm.at[pl.ds(hb + k * CHUNK, rn)], ss[k % 2])
        pltpu.sync_copy(den_s.at[pl.ds(base + k * CHUNK, rn)],
                        eb.at[0, pl.ds(0, rn)])
        pltpu.async_copy(eb.at[0, pl.ds(0, rn)],
                         den_hbm.at[pl.ds(hb + k * CHUNK, rn)], sg[k % 2])
    for k in (nzc - 1, nzc):
        rn = CHUNK if k < nzc else rem
        rb = rowf[k % 2]
        eb = ex2 if k % 2 == 0 else bv2
        pltpu.make_async_copy(
            rb.at[pl.ds(0, rn)],
            num_hbm.at[pl.ds(hb + k * CHUNK, rn)], ss[k % 2]).wait()
        pltpu.make_async_copy(
            eb.at[0, pl.ds(0, rn)],
            den_hbm.at[pl.ds(hb + k * CHUNK, rn)], sg[k % 2]).wait()


def _run_edge(h, asrc, adst, cmax, srcA, dstA):
    h32 = lax.bitcast_convert_type(h.reshape(N, D // 2, 2), jnp.int32)
    num, den = _edge(h32, srcA, dstA, asrc.reshape(N), adst.reshape(N), cmax)
    return (num[:N], num[NPAD:NPAD + N],
            den[:N].reshape(N, 1), den[NPAD:NPAD + N].reshape(N, 1))


def kernel(x, edge_index, W1, att_src1, att_dst1, b1,
           W2, att_src2, att_dst2, b2):
    loops = jnp.arange(N, dtype=jnp.int32)
    src = jnp.concatenate([edge_index[0], loops,
                           jnp.zeros((PAD,), jnp.int32)])
    dst = jnp.concatenate([edge_index[1], loops,
                           jnp.full((PAD,), GARBAGE, jnp.int32)])
    srcA = src.reshape(NW, IDXR, 128)
    dstA = dst.reshape(NW, IDXR, 128)

    atts1 = att_src1.reshape(D, 1)
    attd1 = att_dst1.reshape(D, 1)
    atts2 = att_src2.reshape(D, 1)
    attd2 = att_dst2.reshape(D, 1)
    b1_8 = jnp.broadcast_to(b1.reshape(1, D), (8, D))
    b2_8 = jnp.broadcast_to(b2.reshape(1, D), (8, D))

    h1, as1, ad1, c1 = _pre(x, W1, atts1, attd1)
    n0, n1, d0, d1 = _run_edge(h1, as1, ad1, c1, srcA, dstA)
    h2, as2, ad2, c2 = _mid(n0, n1, d0, d1, b1_8, W2, atts2, attd2)
    n0, n1, d0, d1 = _run_edge(h2, as2, ad2, c2, srcA, dstA)
    return _norm(n0, n1, d0, d1, b2_8)


# speedup vs baseline: 1.3089x; 1.3089x over previous
import functools

import jax
import jax.numpy as jnp
from jax import lax
from jax.experimental import pallas as pl
from jax.experimental.pallas import tpu as pltpu
import jax.experimental.pallas.tpu_sc as plsc

N = 10000
D = 128
E = 320000
ETOT = E + N

NW = 32
CHUNK = 64
NCH = 162
IDXR = 81
PER_TILE = NCH * CHUNK
EPAD = NW * PER_TILE
PAD = EPAD - ETOT

NPAD = 10112
GARBAGE = 10016
RPT = NPAD // 16
ZN = 640

BN = 1000
NBLK = N // BN


def _pre_body(act_ref, w_ref, atts_ref, attd_ref,
              h_ref, asrc_ref, adst_ref, c_ref, ms_ref, md_ref):
    i = pl.program_id(0)
    h = jnp.dot(act_ref[...], w_ref[...], preferred_element_type=jnp.float32)
    h_ref[...] = h
    asrc = jnp.dot(h, atts_ref[...], preferred_element_type=jnp.float32)
    adst = jnp.dot(h, attd_ref[...], preferred_element_type=jnp.float32)
    asrc_ref[...] = asrc
    adst_ref[...] = adst

    @pl.when(i == 0)
    def _():
        ms_ref[...] = jnp.full((8, 128), -1e30, jnp.float32)
        md_ref[...] = jnp.full((8, 128), -1e30, jnp.float32)

    ms_ref[...] = jnp.maximum(ms_ref[...], jnp.max(asrc))
    md_ref[...] = jnp.maximum(md_ref[...], jnp.max(adst))

    @pl.when(i == NBLK - 1)
    def _():
        c_ref[...] = jnp.maximum(ms_ref[...] + md_ref[...], 0.0)


def _pre(act, w, atts, attd):
    return pl.pallas_call(
        _pre_body,
        grid=(NBLK,),
        in_specs=[
            pl.BlockSpec((BN, D), lambda i: (i, 0)),
            pl.BlockSpec((D, D), lambda i: (0, 0)),
            pl.BlockSpec((D, 1), lambda i: (0, 0)),
            pl.BlockSpec((D, 1), lambda i: (0, 0)),
        ],
        out_specs=[
            pl.BlockSpec((BN, D), lambda i: (i, 0)),
            pl.BlockSpec((BN, 1), lambda i: (i, 0)),
            pl.BlockSpec((BN, 1), lambda i: (i, 0)),
            pl.BlockSpec((8, 128), lambda i: (0, 0)),
        ],
        out_shape=[
            jax.ShapeDtypeStruct((N, D), jnp.float32),
            jax.ShapeDtypeStruct((N, 1), jnp.float32),
            jax.ShapeDtypeStruct((N, 1), jnp.float32),
            jax.ShapeDtypeStruct((8, 128), jnp.float32),
        ],
        scratch_shapes=[
            pltpu.VMEM((8, 128), jnp.float32),
            pltpu.VMEM((8, 128), jnp.float32),
        ],
    )(act, w, atts, attd)


def _mid_body(n0_ref, n1_ref, d0_ref, d1_ref, b_ref, w_ref, atts_ref, attd_ref,
              h_ref, asrc_ref, adst_ref, c_ref, ms_ref, md_ref):
    i = pl.program_id(0)
    d = d0_ref[...] + d1_ref[...]
    d = jnp.where(d == 0.0, 1.0, d)
    act = (n0_ref[...] + n1_ref[...]) / d + b_ref[...][0:1, :]
    act = jnp.maximum(act, 0.0)
    h = jnp.dot(act, w_ref[...], preferred_element_type=jnp.float32)
    h_ref[...] = h
    asrc = jnp.dot(h, atts_ref[...], preferred_element_type=jnp.float32)
    adst = jnp.dot(h, attd_ref[...], preferred_element_type=jnp.float32)
    asrc_ref[...] = asrc
    adst_ref[...] = adst

    @pl.when(i == 0)
    def _():
        ms_ref[...] = jnp.full((8, 128), -1e30, jnp.float32)
        md_ref[...] = jnp.full((8, 128), -1e30, jnp.float32)

    ms_ref[...] = jnp.maximum(ms_ref[...], jnp.max(asrc))
    md_ref[...] = jnp.maximum(md_ref[...], jnp.max(adst))

    @pl.when(i == NBLK - 1)
    def _():
        c_ref[...] = jnp.maximum(ms_ref[...] + md_ref[...], 0.0)


def _mid(n0, n1, d0, d1, b8, w, atts, attd):
    return pl.pallas_call(
        _mid_body,
        grid=(NBLK,),
        in_specs=[
            pl.BlockSpec((BN, D), lambda i: (i, 0)),
            pl.BlockSpec((BN, D), lambda i: (i, 0)),
            pl.BlockSpec((BN, 1), lambda i: (i, 0)),
            pl.BlockSpec((BN, 1), lambda i: (i, 0)),
            pl.BlockSpec((8, 128), lambda i: (0, 0)),
            pl.BlockSpec((D, D), lambda i: (0, 0)),
            pl.BlockSpec((D, 1), lambda i: (0, 0)),
            pl.BlockSpec((D, 1), lambda i: (0, 0)),
        ],
        out_specs=[
            pl.BlockSpec((BN, D), lambda i: (i, 0)),
            pl.BlockSpec((BN, 1), lambda i: (i, 0)),
            pl.BlockSpec((BN, 1), lambda i: (i, 0)),
            pl.BlockSpec((8, 128), lambda i: (0, 0)),
        ],
        out_shape=[
            jax.ShapeDtypeStruct((N, D), jnp.float32),
            jax.ShapeDtypeStruct((N, 1), jnp.float32),
            jax.ShapeDtypeStruct((N, 1), jnp.float32),
            jax.ShapeDtypeStruct((8, 128), jnp.float32),
        ],
        scratch_shapes=[
            pltpu.VMEM((8, 128), jnp.float32),
            pltpu.VMEM((8, 128), jnp.float32),
        ],
    )(n0, n1, d0, d1, b8, w, atts, attd)


def _norm_body(n0_ref, n1_ref, d0_ref, d1_ref, b_ref, out_ref):
    d = d0_ref[...] + d1_ref[...]
    d = jnp.where(d == 0.0, 1.0, d)
    a = (n0_ref[...] + n1_ref[...]) / d + b_ref[...][0:1, :]
    out_ref[...] = jnp.maximum(a, 0.0)


def _norm(n0, n1, d0, d1, b8):
    return pl.pallas_call(
        _norm_body,
        grid=(NBLK,),
        in_specs=[
            pl.BlockSpec((BN, D), lambda i: (i, 0)),
            pl.BlockSpec((BN, D), lambda i: (i, 0)),
            pl.BlockSpec((BN, 1), lambda i: (i, 0)),
            pl.BlockSpec((BN, 1), lambda i: (i, 0)),
            pl.BlockSpec((8, 128), lambda i: (0, 0)),
        ],
        out_specs=pl.BlockSpec((BN, D), lambda i: (i, 0)),
        out_shape=jax.ShapeDtypeStruct((N, D), jnp.float32),
    )(n0, n1, d0, d1, b8)


_MESH = plsc.VectorSubcoreMesh(core_axis_name="c", subcore_axis_name="s")


@functools.partial(
    pl.kernel,
    out_type=[
        jax.ShapeDtypeStruct((2 * NPAD, D), jnp.float32),
        jax.ShapeDtypeStruct((2 * NPAD,), jnp.float32),
    ],
    mesh=_MESH,
    scratch_types=[
        pltpu.VMEM_SHARED((NPAD, D), jnp.float32),
        pltpu.VMEM_SHARED((NPAD,), jnp.float32),
        pltpu.VMEM((IDXR, 128), jnp.int32),
        pltpu.VMEM((IDXR, 128), jnp.int32),
        pltpu.VMEM((3, CHUNK), jnp.int32),
        pltpu.VMEM((3, CHUNK), jnp.float32),
        pltpu.VMEM((3, CHUNK), jnp.float32),
        pltpu.VMEM((3, CHUNK), jnp.float32),
        pltpu.VMEM((128,), jnp.float32),
        pltpu.VMEM((CHUNK, D), jnp.float32),
        pltpu.VMEM((CHUNK, D), jnp.float32),
        pltpu.VMEM((CHUNK, D), jnp.float32),
        pltpu.SemaphoreType.DMA,
        pltpu.SemaphoreType.DMA,
        pltpu.SemaphoreType.DMA,
        pltpu.SemaphoreType.DMA,
        pltpu.SemaphoreType.DMA,
        pltpu.SemaphoreType.DMA,
    ],
    compiler_params=pltpu.CompilerParams(needs_layout_passes=False),
)
def _edge(h_hbm, src_hbm, dst_hbm, asrc_hbm, adst_hbm, c_hbm,
          num_hbm, den_hbm,
          num_s, den_s, src2d, dst2d, didx3, ex3, av3, bv3, cbuf,
          rows0, rows1, rows2,
          sg0, sg1, sg2, ss0, ss1, ss2):
    c = lax.axis_index("c")
    s = lax.axis_index("s")
    w = c * 16 + s
    rows = [rows0, rows1, rows2]
    sg = [sg0, sg1, sg2]
    ss = [ss0, ss1, ss2]

    pltpu.async_copy(src_hbm.at[w], src2d, sg1)
    pltpu.async_copy(dst_hbm.at[w], dst2d, sg1)
    pltpu.async_copy(c_hbm.at[0], cbuf, sg1)

    def _zrow(r, _):
        for j in range(8):
            rows0[r, pl.ds(j * 16, 16)] = jnp.zeros((16,), jnp.float32)
        return 0
    lax.fori_loop(0, CHUNK, _zrow, 0)
    for g in range(CHUNK // 16):
        av3[0, pl.ds(g * 16, 16)] = jnp.zeros((16,), jnp.float32)

    base = s * RPT
    nzc = RPT // CHUNK
    rem = RPT - nzc * CHUNK
    for k in range(nzc):
        pltpu.async_copy(rows0, num_s.at[pl.ds(base + k * CHUNK, CHUNK)], sg0)
        pltpu.async_copy(av3.at[0], den_s.at[pl.ds(base + k * CHUNK, CHUNK)],
                         sg0)
    pltpu.async_copy(rows0.at[pl.ds(0, rem)],
                     num_s.at[pl.ds(base + nzc * CHUNK, rem)], sg0)
    pltpu.async_copy(av3.at[0, pl.ds(0, rem)],
                     den_s.at[pl.ds(base + nzc * CHUNK, rem)], sg0)
    for k in range(nzc):
        pltpu.make_async_copy(
            rows0, num_s.at[pl.ds(base + k * CHUNK, CHUNK)], sg0).wait()
        pltpu.make_async_copy(
            av3.at[0], den_s.at[pl.ds(base + k * CHUNK, CHUNK)], sg0).wait()
    pltpu.make_async_copy(
        rows0.at[pl.ds(0, rem)],
        num_s.at[pl.ds(base + nzc * CHUNK, rem)], sg0).wait()
    pltpu.make_async_copy(
        av3.at[0, pl.ds(0, rem)],
        den_s.at[pl.ds(base + nzc * CHUNK, rem)], sg0).wait()
    pltpu.make_async_copy(src_hbm.at[w], src2d, sg1).wait()
    pltpu.make_async_copy(dst_hbm.at[w], dst2d, sg1).wait()
    pltpu.make_async_copy(c_hbm.at[0], cbuf, sg1).wait()
    plsc.subcore_barrier()

    cvec = cbuf[pl.ds(0, 16)]

    def _sidx(jh, half):
        return src2d.at[jh, pl.ds(half * CHUNK, CHUNK)]

    def _didx(jh, half):
        return dst2d.at[jh, pl.ds(half * CHUNK, CHUNK)]

    def _issue(jh, half, b):
        pltpu.async_copy(h_hbm.at[_sidx(jh, half)], rows[b], sg[b])
        pltpu.async_copy(asrc_hbm.at[_sidx(jh, half)], av3.at[b], sg[b])
        pltpu.async_copy(adst_hbm.at[_didx(jh, half)], bv3.at[b], sg[b])

    def _wait_gathers(b):
        pltpu.make_async_copy(h_hbm.at[_sidx(0, 0)], rows[b], sg[b]).wait()
        pltpu.make_async_copy(asrc_hbm.at[_sidx(0, 0)], av3.at[b], sg[b]).wait()
        pltpu.make_async_copy(adst_hbm.at[_didx(0, 0)], bv3.at[b], sg[b]).wait()

    def _wait_scatters(b):
        pltpu.make_async_copy(ex3.at[b], den_s.at[didx3.at[0]], ss[b]).wait()
        pltpu.make_async_copy(rows[b], num_s.at[didx3.at[0]], ss[b]).wait()

    def _consume(jh, half, b):
        _wait_gathers(b)
        for g in range(CHUNK // 16):
            d16 = dst2d[jh, pl.ds(half * CHUNK + g * 16, 16)]
            didx3[b, pl.ds(g * 16, 16)] = d16
            avv = av3[b, pl.ds(g * 16, 16)]
            bvv = bv3[b, pl.ds(g * 16, 16)]
            al = avv + bvv
            al = jnp.maximum(al, al * jnp.float32(0.2))
            ex3[b, pl.ds(g * 16, 16)] = jnp.exp(al - cvec)
        pltpu.async_copy(ex3.at[b], den_s.at[didx3.at[b]], ss[b], add=True)

        rb = rows[b]
        def _scale(r4, _):
            r0 = r4 * 4
            ebs = [plsc.load_gather(
                ex3, [jnp.full((16,), b, jnp.int32),
                      jnp.full((16,), r0 + u, jnp.int32)]) for u in range(4)]
            for u in range(4):
                for j2 in range(8):
                    rb[r0 + u, pl.ds(j2 * 16, 16)] = (
                        rb[r0 + u, pl.ds(j2 * 16, 16)] * ebs[u])
            return 0
        lax.fori_loop(0, CHUNK // 4, _scale, 0)
        pltpu.async_copy(rb, num_s.at[didx3.at[b]], ss[b], add=True)

    _issue(0, 0, 0)
    _issue(0, 1, 1)

    def _superstep(k, _):
        for t in range(6):
            b = t % 3
            half = t % 2
            jh = 3 * k + (t // 2)
            j = 6 * k + t
            _consume(jh, half, b)
            bn = (b + 2) % 3
            tn = t + 2
            jhn = 3 * k + ((tn % 6) // 2) + (tn // 6) * 3
            halfn = tn % 2
            @pl.when(j >= 1)
            def _():
                _wait_scatters(bn)
            @pl.when(j + 2 < NCH)
            def _():
                _issue(jhn, halfn, bn)
        return 0
    lax.fori_loop(0, NCH // 6, _superstep, 0)

    _wait_scatters((NCH - 1) % 3)
    plsc.subcore_barrier()

    hb = c * NPAD + base
    for k in range(nzc + 1):
        rn = CHUNK if k < nzc else rem
        rb = rows[k % 2]
        eb = ex3 if k % 2 == 0 else bv3
        if k >= 2:
            pltpu.make_async_copy(
                rb, num_hbm.at[pl.ds(hb + (k - 2) * CHUNK, CHUNK)],
                ss[k % 2]).wait()
            pltpu.make_async_copy(
                eb.at[0], den_hbm.at[pl.ds(hb + (k - 2) * CHUNK, CHUNK)],
                sg[k % 2]).wait()
        pltpu.sync_copy(num_s.at[pl.ds(base + k * CHUNK, rn)],
                        rb.at[pl.ds(0, rn)])
        pltpu.async_copy(rb.at[pl.ds(0, rn)],
                         num_hbm.at[pl.ds(hb + k * CHUNK, rn)], ss[k % 2])
        pltpu.sync_copy(den_s.at[pl.ds(base + k * CHUNK, rn)],
                        eb.at[0, pl.ds(0, rn)])
        pltpu.async_copy(eb.at[0, pl.ds(0, rn)],
                         den_hbm.at[pl.ds(hb + k * CHUNK, rn)], sg[k % 2])
    for k in (nzc - 1, nzc):
        rn = CHUNK if k < nzc else rem
        rb = rows[k % 2]
        eb = ex3 if k % 2 == 0 else bv3
        pltpu.make_async_copy(
            rb.at[pl.ds(0, rn)],
            num_hbm.at[pl.ds(hb + k * CHUNK, rn)], ss[k % 2]).wait()
        pltpu.make_async_copy(
            eb.at[0, pl.ds(0, rn)],
            den_hbm.at[pl.ds(hb + k * CHUNK, rn)], sg[k % 2]).wait()


def _run_edge(h, asrc, adst, cmax, srcA, dstA):
    num, den = _edge(h, srcA, dstA, asrc.reshape(N), adst.reshape(N), cmax)
    return (num[:N], num[NPAD:NPAD + N],
            den[:N].reshape(N, 1), den[NPAD:NPAD + N].reshape(N, 1))


def kernel(x, edge_index, W1, att_src1, att_dst1, b1,
           W2, att_src2, att_dst2, b2):
    loops = jnp.arange(N, dtype=jnp.int32)
    src = jnp.concatenate([edge_index[0], loops,
                           jnp.zeros((PAD,), jnp.int32)])
    dst = jnp.concatenate([edge_index[1], loops,
                           jnp.full((PAD,), GARBAGE, jnp.int32)])
    srcA = src.reshape(NW, IDXR, 128)
    dstA = dst.reshape(NW, IDXR, 128)

    atts1 = att_src1.reshape(D, 1)
    attd1 = att_dst1.reshape(D, 1)
    atts2 = att_src2.reshape(D, 1)
    attd2 = att_dst2.reshape(D, 1)
    b1_8 = jnp.broadcast_to(b1.reshape(1, D), (8, D))
    b2_8 = jnp.broadcast_to(b2.reshape(1, D), (8, D))

    h1, as1, ad1, c1 = _pre(x, W1, atts1, attd1)
    n0, n1, d0, d1 = _run_edge(h1, as1, ad1, c1, srcA, dstA)
    h2, as2, ad2, c2 = _mid(n0, n1, d0, d1, b1_8, W2, atts2, attd2)
    n0, n1, d0, d1 = _run_edge(h2, as2, ad2, c2, srcA, dstA)
    return _norm(n0, n1, d0, d1, b2_8)
